# baseline (device time: 37087 ns/iter reference)
import jax
import jax.numpy as jnp
from jax import lax
from jax.experimental import pallas as pl
from jax.experimental.pallas import tpu as pltpu

N_DEV = 4
N_TOK = 512
D_IN = 256
D_OUT = 512
N_EXP = 16
E_LOCAL = N_EXP // N_DEV
CHUNK = N_TOK // N_DEV


def _mod(v):
    return lax.rem(v + 2 * N_DEV, N_DEV)


def kernel(x, router_W, route_idx, expert_W):
    def body(x_ref, rw_ref, idx_ref, ew_ref, out_ref,
             comm_ref, rs_send, rs_recv, ag_send, ag_recv):
        my = lax.axis_index("i")
        left = _mod(my - 1)
        right = _mod(my + 1)

        barrier_sem = pltpu.get_barrier_semaphore()
        for nbr in [left, right]:
            pl.semaphore_signal(
                barrier_sem, inc=1,
                device_id=(nbr,), device_id_type=pl.DeviceIdType.MESH,
            )
        pl.semaphore_wait(barrier_sem, 2)

        xf = x_ref[:, :]
        scores = jnp.dot(xf, rw_ref[:, :], preferred_element_type=jnp.float32)
        s_max = jnp.max(scores, axis=-1, keepdims=True)
        p = jnp.exp(scores - s_max)
        probs = p / jnp.sum(p, axis=-1, keepdims=True)

        idx0 = idx_ref[:, 0:1]
        idx1 = idx_ref[:, 1:2]
        eids = lax.broadcasted_iota(jnp.int32, (N_TOK, N_EXP), 1)
        g0 = jnp.sum(jnp.where(eids == idx0, probs, 0.0), axis=1, keepdims=True)
        g1 = jnp.sum(jnp.where(eids == idx1, probs, 0.0), axis=1, keepdims=True)
        gs = g0 + g1
        g0 = g0 / gs
        g1 = g1 / gs

        acc = jnp.zeros((N_TOK, D_OUT), jnp.float32)
        for j in range(E_LOCAL):
            e = my * E_LOCAL + j
            w = (jnp.where(idx0 == e, g0, 0.0)
                 + jnp.where(idx1 == e, g1, 0.0))
            xw = (w * xf).astype(jnp.bfloat16)
            acc = acc + jnp.dot(xw, ew_ref[j].astype(jnp.bfloat16),
                                preferred_element_type=jnp.float32)
        out_ref[:, :] = acc

        for s in range(N_DEV - 1):
            send_c = _mod(my - s)
            recv_c = _mod(my - s - 1)
            rdma = pltpu.make_async_remote_copy(
                src_ref=out_ref.at[pl.ds(send_c * CHUNK, CHUNK), :],
                dst_ref=comm_ref.at[s],
                send_sem=rs_send.at[s],
                recv_sem=rs_recv.at[s],
                device_id=(right,),
                device_id_type=pl.DeviceIdType.MESH,
            )
            rdma.start()
            rdma.wait()
            out_ref[pl.ds(recv_c * CHUNK, CHUNK), :] = (
                out_ref[pl.ds(recv_c * CHUNK, CHUNK), :] + comm_ref[s]
            )

        for h in range(N_DEV - 1):
            send_c = _mod(my + 1 - h)
            rdma = pltpu.make_async_remote_copy(
                src_ref=out_ref.at[pl.ds(send_c * CHUNK, CHUNK), :],
                dst_ref=out_ref.at[pl.ds(send_c * CHUNK, CHUNK), :],
                send_sem=ag_send.at[h],
                recv_sem=ag_recv.at[h],
                device_id=(right,),
                device_id_type=pl.DeviceIdType.MESH,
            )
            rdma.start()
            rdma.wait()

    return pl.pallas_call(
        body,
        out_shape=jax.ShapeDtypeStruct((N_TOK, D_OUT), jnp.float32),
        in_specs=[
            pl.BlockSpec(memory_space=pltpu.VMEM),
            pl.BlockSpec(memory_space=pltpu.VMEM),
            pl.BlockSpec(memory_space=pltpu.VMEM),
            pl.BlockSpec(memory_space=pltpu.VMEM),
        ],
        out_specs=pl.BlockSpec(memory_space=pltpu.VMEM),
        scratch_shapes=[
            pltpu.VMEM((N_DEV - 1, CHUNK, D_OUT), jnp.float32),
            pltpu.SemaphoreType.DMA((N_DEV - 1,)),
            pltpu.SemaphoreType.DMA((N_DEV - 1,)),
            pltpu.SemaphoreType.DMA((N_DEV - 1,)),
            pltpu.SemaphoreType.DMA((N_DEV - 1,)),
        ],
        compiler_params=pltpu.CompilerParams(collective_id=0),
    )(x, router_W, route_idx, expert_W)


# device time: 20218 ns/iter; 1.8344x vs baseline; 1.8344x over previous
import jax
import jax.numpy as jnp
from jax import lax
from jax.experimental import pallas as pl
from jax.experimental.pallas import tpu as pltpu

N_DEV = 4
N_TOK = 512
D_IN = 256
D_OUT = 512
N_EXP = 16
E_LOCAL = N_EXP // N_DEV
CHUNK = N_TOK // N_DEV


def _mod(v):
    return lax.rem(v + 2 * N_DEV, N_DEV)


def kernel(x, router_W, route_idx, expert_W):
    def body(x_ref, rw_ref, idx_ref, ew_ref, out_ref,
             rs_send_buf, rs_recv_buf, ag_send_buf, ag_recv_buf,
             rs_send_sems, rs_recv_sems, ag_send_sems, ag_recv_sems):
        my = lax.axis_index("i")
        peers = [_mod(my + k + 1) for k in range(N_DEV - 1)]

        barrier_sem = pltpu.get_barrier_semaphore()
        for q in peers:
            pl.semaphore_signal(
                barrier_sem, inc=1,
                device_id=(q,), device_id_type=pl.DeviceIdType.MESH,
            )
        pl.semaphore_wait(barrier_sem, N_DEV - 1)

        xf = x_ref[:, :]
        scores = jnp.dot(xf, rw_ref[:, :], preferred_element_type=jnp.float32)
        s_max = jnp.max(scores, axis=-1, keepdims=True)
        p = jnp.exp(scores - s_max)
        probs = p / jnp.sum(p, axis=-1, keepdims=True)

        idx0 = idx_ref[:, 0:1]
        idx1 = idx_ref[:, 1:2]
        eids = lax.broadcasted_iota(jnp.int32, (N_TOK, N_EXP), 1)
        g0 = jnp.sum(jnp.where(eids == idx0, probs, 0.0), axis=1, keepdims=True)
        g1 = jnp.sum(jnp.where(eids == idx1, probs, 0.0), axis=1, keepdims=True)
        gs = g0 + g1
        g0 = g0 / gs
        g1 = g1 / gs

        acc = jnp.zeros((N_TOK, D_OUT), jnp.float32)
        for j in range(E_LOCAL):
            e = my * E_LOCAL + j
            w = (jnp.where(idx0 == e, g0, 0.0)
                 + jnp.where(idx1 == e, g1, 0.0))
            xw = (w * xf).astype(jnp.bfloat16)
            acc = acc + jnp.dot(xw, ew_ref[j].astype(jnp.bfloat16),
                                preferred_element_type=jnp.float32)
        out_ref[:, :] = acc

        rs = []
        for k, q in enumerate(peers):
            rs_send_buf[k] = (
                out_ref[pl.ds(q * CHUNK, CHUNK), :].astype(jnp.bfloat16)
            )
            r = pltpu.make_async_remote_copy(
                src_ref=rs_send_buf.at[k],
                dst_ref=rs_recv_buf.at[2 - k],
                send_sem=rs_send_sems.at[k],
                recv_sem=rs_recv_sems.at[2 - k],
                device_id=(q,),
                device_id_type=pl.DeviceIdType.MESH,
            )
            r.start()
            rs.append(r)
        for j in range(N_DEV - 1):
            pltpu.make_async_remote_copy(
                src_ref=rs_send_buf.at[j],
                dst_ref=rs_recv_buf.at[j],
                send_sem=rs_send_sems.at[j],
                recv_sem=rs_recv_sems.at[j],
                device_id=(my,),
                device_id_type=pl.DeviceIdType.MESH,
            ).wait_recv()

        red = out_ref[pl.ds(my * CHUNK, CHUNK), :]
        for j in range(N_DEV - 1):
            red = red + rs_recv_buf[j].astype(jnp.float32)
        out_ref[pl.ds(my * CHUNK, CHUNK), :] = red
        ag_send_buf[:, :] = red.astype(jnp.bfloat16)
        for r in rs:
            r.wait_send()

        ag = []
        for k, q in enumerate(peers):
            r = pltpu.make_async_remote_copy(
                src_ref=ag_send_buf,
                dst_ref=ag_recv_buf.at[2 - k],
                send_sem=ag_send_sems.at[k],
                recv_sem=ag_recv_sems.at[2 - k],
                device_id=(q,),
                device_id_type=pl.DeviceIdType.MESH,
            )
            r.start()
            ag.append(r)
        for j in range(N_DEV - 1):
            u = _mod(my + j + 1)
            pltpu.make_async_remote_copy(
                src_ref=ag_send_buf,
                dst_ref=ag_recv_buf.at[j],
                send_sem=ag_send_sems.at[j],
                recv_sem=ag_recv_sems.at[j],
                device_id=(my,),
                device_id_type=pl.DeviceIdType.MESH,
            ).wait_recv()
            out_ref[pl.ds(u * CHUNK, CHUNK), :] = (
                ag_recv_buf[j].astype(jnp.float32)
            )
        for r in ag:
            r.wait_send()

    return pl.pallas_call(
        body,
        out_shape=jax.ShapeDtypeStruct((N_TOK, D_OUT), jnp.float32),
        in_specs=[
            pl.BlockSpec(memory_space=pltpu.VMEM),
            pl.BlockSpec(memory_space=pltpu.VMEM),
            pl.BlockSpec(memory_space=pltpu.VMEM),
            pl.BlockSpec(memory_space=pltpu.VMEM),
        ],
        out_specs=pl.BlockSpec(memory_space=pltpu.VMEM),
        scratch_shapes=[
            pltpu.VMEM((N_DEV - 1, CHUNK, D_OUT), jnp.bfloat16),
            pltpu.VMEM((N_DEV - 1, CHUNK, D_OUT), jnp.bfloat16),
            pltpu.VMEM((CHUNK, D_OUT), jnp.bfloat16),
            pltpu.VMEM((N_DEV - 1, CHUNK, D_OUT), jnp.bfloat16),
            pltpu.SemaphoreType.DMA((N_DEV - 1,)),
            pltpu.SemaphoreType.DMA((N_DEV - 1,)),
            pltpu.SemaphoreType.DMA((N_DEV - 1,)),
            pltpu.SemaphoreType.DMA((N_DEV - 1,)),
        ],
        compiler_params=pltpu.CompilerParams(collective_id=0),
    )(x, router_W, route_idx, expert_W)


# device time: 20073 ns/iter; 1.8476x vs baseline; 1.0072x over previous
import jax
import jax.numpy as jnp
from jax import lax
from jax.experimental import pallas as pl
from jax.experimental.pallas import tpu as pltpu

N_DEV = 4
N_TOK = 512
D_IN = 256
D_OUT = 512
N_EXP = 16
E_LOCAL = N_EXP // N_DEV
CHUNK = N_TOK // N_DEV


def _mod(v):
    return lax.rem(v + 2 * N_DEV, N_DEV)


def kernel(x, router_W, route_idx, expert_W):
    def body(x_ref, rw_ref, idx_ref, ew_ref, out_ref,
             ewb_ref, w_ref, rs_send_buf, rs_recv_buf, ag_send_buf,
             ag_recv_buf,
             rs_send_sems, rs_recv_sems, ag_send_sems, ag_recv_sems):
        my = lax.axis_index("i")
        peers = [_mod(my + k + 1) for k in range(N_DEV - 1)]

        barrier_sem = pltpu.get_barrier_semaphore()
        for q in peers:
            pl.semaphore_signal(
                barrier_sem, inc=1,
                device_id=(q,), device_id_type=pl.DeviceIdType.MESH,
            )
        pl.semaphore_wait(barrier_sem, N_DEV - 1)

        xf = x_ref[:, :]
        scores = jnp.dot(xf, rw_ref[:, :], preferred_element_type=jnp.float32)
        s_max = jnp.max(scores, axis=-1, keepdims=True)
        p = jnp.exp(scores - s_max)
        probs = p / jnp.sum(p, axis=-1, keepdims=True)

        idx0 = idx_ref[:, 0:1]
        idx1 = idx_ref[:, 1:2]
        eids = lax.broadcasted_iota(jnp.int32, (N_TOK, N_EXP), 1)
        g0 = jnp.sum(jnp.where(eids == idx0, probs, 0.0), axis=1, keepdims=True)
        g1 = jnp.sum(jnp.where(eids == idx1, probs, 0.0), axis=1, keepdims=True)
        gs = g0 + g1
        g0 = g0 / gs
        g1 = g1 / gs
        w_cols = []
        for j in range(E_LOCAL):
            e = my * E_LOCAL + j
            w_cols.append(jnp.where(idx0 == e, g0, 0.0)
                          + jnp.where(idx1 == e, g1, 0.0))
        w_ref[:, :] = jnp.concatenate(w_cols, axis=1)

        ewb_ref[...] = ew_ref[...].astype(jnp.bfloat16)
        ewcat = ewb_ref[...].reshape(E_LOCAL * D_IN, D_OUT)

        def partial_chunk(c):
            rows = pl.ds(c * CHUNK, CHUNK)
            xc = x_ref[rows, :]
            wc = w_ref[rows, :]
            xcat = jnp.concatenate(
                [(wc[:, j:j + 1] * xc).astype(jnp.bfloat16)
                 for j in range(E_LOCAL)],
                axis=1,
            )
            return jnp.dot(xcat, ewcat, preferred_element_type=jnp.float32)

        rs = []
        for k, q in enumerate(peers):
            rs_send_buf[k] = partial_chunk(q).astype(jnp.bfloat16)
            r = pltpu.make_async_remote_copy(
                src_ref=rs_send_buf.at[k],
                dst_ref=rs_recv_buf.at[2 - k],
                send_sem=rs_send_sems.at[k],
                recv_sem=rs_recv_sems.at[2 - k],
                device_id=(q,),
                device_id_type=pl.DeviceIdType.MESH,
            )
            r.start()
            rs.append(r)

        red = partial_chunk(my)

        for j in range(N_DEV - 1):
            pltpu.make_async_remote_copy(
                src_ref=rs_send_buf.at[j],
                dst_ref=rs_recv_buf.at[j],
                send_sem=rs_send_sems.at[j],
                recv_sem=rs_recv_sems.at[j],
                device_id=(my,),
                device_id_type=pl.DeviceIdType.MESH,
            ).wait_recv()
        for j in range(N_DEV - 1):
            red = red + rs_recv_buf[j].astype(jnp.float32)
        out_ref[pl.ds(my * CHUNK, CHUNK), :] = red
        ag_send_buf[:, :] = red.astype(jnp.bfloat16)
        for r in rs:
            r.wait_send()

        ag = []
        for k, q in enumerate(peers):
            r = pltpu.make_async_remote_copy(
                src_ref=ag_send_buf,
                dst_ref=ag_recv_buf.at[2 - k],
                send_sem=ag_send_sems.at[k],
                recv_sem=ag_recv_sems.at[2 - k],
                device_id=(q,),
                device_id_type=pl.DeviceIdType.MESH,
            )
            r.start()
            ag.append(r)
        for j in range(N_DEV - 1):
            u = _mod(my + j + 1)
            pltpu.make_async_remote_copy(
                src_ref=ag_send_buf,
                dst_ref=ag_recv_buf.at[j],
                send_sem=ag_send_sems.at[j],
                recv_sem=ag_recv_sems.at[j],
                device_id=(my,),
                device_id_type=pl.DeviceIdType.MESH,
            ).wait_recv()
            out_ref[pl.ds(u * CHUNK, CHUNK), :] = (
                ag_recv_buf[j].astype(jnp.float32)
            )
        for r in ag:
            r.wait_send()

    return pl.pallas_call(
        body,
        out_shape=jax.ShapeDtypeStruct((N_TOK, D_OUT), jnp.float32),
        in_specs=[
            pl.BlockSpec(memory_space=pltpu.VMEM),
            pl.BlockSpec(memory_space=pltpu.VMEM),
            pl.BlockSpec(memory_space=pltpu.VMEM),
            pl.BlockSpec(memory_space=pltpu.VMEM),
        ],
        out_specs=pl.BlockSpec(memory_space=pltpu.VMEM),
        scratch_shapes=[
            pltpu.VMEM((E_LOCAL, D_IN, D_OUT), jnp.bfloat16),
            pltpu.VMEM((N_TOK, E_LOCAL), jnp.float32),
            pltpu.VMEM((N_DEV - 1, CHUNK, D_OUT), jnp.bfloat16),
            pltpu.VMEM((N_DEV - 1, CHUNK, D_OUT), jnp.bfloat16),
            pltpu.VMEM((CHUNK, D_OUT), jnp.bfloat16),
            pltpu.VMEM((N_DEV - 1, CHUNK, D_OUT), jnp.bfloat16),
            pltpu.SemaphoreType.DMA((N_DEV - 1,)),
            pltpu.SemaphoreType.DMA((N_DEV - 1,)),
            pltpu.SemaphoreType.DMA((N_DEV - 1,)),
            pltpu.SemaphoreType.DMA((N_DEV - 1,)),
        ],
        compiler_params=pltpu.CompilerParams(collective_id=0),
    )(x, router_W, route_idx, expert_W)


# device time: 19154 ns/iter; 1.9363x vs baseline; 1.0480x over previous
import jax
import jax.numpy as jnp
from jax import lax
from jax.experimental import pallas as pl
from jax.experimental.pallas import tpu as pltpu

N_DEV = 4
N_TOK = 512
D_IN = 256
D_OUT = 512
N_EXP = 16
E_LOCAL = N_EXP // N_DEV
CHUNK = N_TOK // N_DEV
HALF = CHUNK // 2
N_PEER = N_DEV - 1


def _mod(v):
    return lax.rem(v + 2 * N_DEV, N_DEV)


def kernel(x, router_W, route_idx, expert_W):
    def body(x_ref, rw_ref, idx_ref, ew_ref, out_ref,
             ewb_ref, w_ref, rs_send_buf, rs_recv_buf, ag_send_buf,
             ag_recv_buf,
             rs_send_sems, rs_recv_sems, ag_send_sems, ag_recv_sems):
        my = lax.axis_index("i")
        peers = [_mod(my + k + 1) for k in range(N_PEER)]

        barrier_sem = pltpu.get_barrier_semaphore()
        for q in peers:
            pl.semaphore_signal(
                barrier_sem, inc=1,
                device_id=(q,), device_id_type=pl.DeviceIdType.MESH,
            )
        pl.semaphore_wait(barrier_sem, N_PEER)

        xf = x_ref[:, :]
        scores = jnp.dot(xf, rw_ref[:, :], preferred_element_type=jnp.float32)
        s_max = jnp.max(scores, axis=-1, keepdims=True)
        p = jnp.exp(scores - s_max)
        probs = p / jnp.sum(p, axis=-1, keepdims=True)

        idx0 = idx_ref[:, 0:1]
        idx1 = idx_ref[:, 1:2]
        eids = lax.broadcasted_iota(jnp.int32, (N_TOK, N_EXP), 1)
        g0 = jnp.sum(jnp.where(eids == idx0, probs, 0.0), axis=1, keepdims=True)
        g1 = jnp.sum(jnp.where(eids == idx1, probs, 0.0), axis=1, keepdims=True)
        gs = g0 + g1
        g0 = g0 / gs
        g1 = g1 / gs
        w_cols = []
        for j in range(E_LOCAL):
            e = my * E_LOCAL + j
            w_cols.append(jnp.where(idx0 == e, g0, 0.0)
                          + jnp.where(idx1 == e, g1, 0.0))
        w_ref[:, :] = jnp.concatenate(w_cols, axis=1)

        ewb_ref[...] = ew_ref[...].astype(jnp.bfloat16)
        ewcat = ewb_ref[...].reshape(E_LOCAL * D_IN, D_OUT)

        def partial_chunk(c):
            rows = pl.ds(c * CHUNK, CHUNK)
            xc = x_ref[rows, :]
            wc = w_ref[rows, :]
            xcat = jnp.concatenate(
                [(wc[:, j:j + 1] * xc).astype(jnp.bfloat16)
                 for j in range(E_LOCAL)],
                axis=1,
            )
            return jnp.dot(xcat, ewcat, preferred_element_type=jnp.float32)

        def rdma(src, dst, ssem, rsem, dev):
            return pltpu.make_async_remote_copy(
                src_ref=src, dst_ref=dst, send_sem=ssem, recv_sem=rsem,
                device_id=(dev,), device_id_type=pl.DeviceIdType.MESH,
            )

        rs = []
        for k, q in enumerate(peers):
            rs_send_buf[k] = partial_chunk(q).astype(jnp.bfloat16)
            for h in range(2):
                r = rdma(rs_send_buf.at[k, pl.ds(h * HALF, HALF), :],
                         rs_recv_buf.at[2 - k, pl.ds(h * HALF, HALF), :],
                         rs_send_sems.at[2 * k + h],
                         rs_recv_sems.at[2 * (2 - k) + h],
                         q)
                r.start()
                rs.append(r)

        mine = partial_chunk(my)

        ag = []
        for h in range(2):
            for j in range(N_PEER):
                rdma(rs_send_buf.at[j, pl.ds(h * HALF, HALF), :],
                     rs_recv_buf.at[j, pl.ds(h * HALF, HALF), :],
                     rs_send_sems.at[2 * j + h],
                     rs_recv_sems.at[2 * j + h],
                     my).wait_recv()
            red = mine[h * HALF:(h + 1) * HALF, :]
            for j in range(N_PEER):
                red = red + rs_recv_buf[j, pl.ds(h * HALF, HALF), :].astype(
                    jnp.float32)
            out_ref[pl.ds(my * CHUNK + h * HALF, HALF), :] = red
            ag_send_buf[h] = red.astype(jnp.bfloat16)
            for k, q in enumerate(peers):
                r = rdma(ag_send_buf.at[h],
                         ag_recv_buf.at[2 - k, h],
                         ag_send_sems.at[2 * k + h],
                         ag_recv_sems.at[2 * (2 - k) + h],
                         q)
                r.start()
                ag.append(r)
        for r in rs:
            r.wait_send()

        for j in range(N_PEER):
            u = _mod(my + j + 1)
            for h in range(2):
                rdma(ag_send_buf.at[h],
                     ag_recv_buf.at[j, h],
                     ag_send_sems.at[2 * j + h],
                     ag_recv_sems.at[2 * j + h],
                     my).wait_recv()
                out_ref[pl.ds(u * CHUNK + h * HALF, HALF), :] = (
                    ag_recv_buf[j, h].astype(jnp.float32)
                )
        for r in ag:
            r.wait_send()

    return pl.pallas_call(
        body,
        out_shape=jax.ShapeDtypeStruct((N_TOK, D_OUT), jnp.float32),
        in_specs=[
            pl.BlockSpec(memory_space=pltpu.VMEM),
            pl.BlockSpec(memory_space=pltpu.VMEM),
            pl.BlockSpec(memory_space=pltpu.VMEM),
            pl.BlockSpec(memory_space=pltpu.VMEM),
        ],
        out_specs=pl.BlockSpec(memory_space=pltpu.VMEM),
        scratch_shapes=[
            pltpu.VMEM((E_LOCAL, D_IN, D_OUT), jnp.bfloat16),
            pltpu.VMEM((N_TOK, E_LOCAL), jnp.float32),
            pltpu.VMEM((N_PEER, CHUNK, D_OUT), jnp.bfloat16),
            pltpu.VMEM((N_PEER, CHUNK, D_OUT), jnp.bfloat16),
            pltpu.VMEM((2, HALF, D_OUT), jnp.bfloat16),
            pltpu.VMEM((N_PEER, 2, HALF, D_OUT), jnp.bfloat16),
            pltpu.SemaphoreType.DMA((2 * N_PEER,)),
            pltpu.SemaphoreType.DMA((2 * N_PEER,)),
            pltpu.SemaphoreType.DMA((2 * N_PEER,)),
            pltpu.SemaphoreType.DMA((2 * N_PEER,)),
        ],
        compiler_params=pltpu.CompilerParams(collective_id=0),
    )(x, router_W, route_idx, expert_W)


# device time: 18223 ns/iter; 2.0352x vs baseline; 1.0511x over previous
import jax
import jax.numpy as jnp
from jax import lax
from jax.experimental import pallas as pl
from jax.experimental.pallas import tpu as pltpu

N_DEV = 4
N_TOK = 512
D_IN = 256
D_OUT = 512
N_EXP = 16
E_LOCAL = N_EXP // N_DEV
CHUNK = N_TOK // N_DEV
HALF = CHUNK // 2
N_PEER = N_DEV - 1


def _mod(v):
    return lax.rem(v + 2 * N_DEV, N_DEV)


def kernel(x, router_W, route_idx, expert_W):
    def body(x_ref, rw_ref, idx_ref, ew_ref, out_ref,
             ewb_ref, w_ref, rs_send_buf, rs_recv_buf,
             rs_send_sems, rs_recv_sems, ag_send_sems, ag_recv_sems):
        my = lax.axis_index("i")
        peers = [_mod(my + k + 1) for k in range(N_PEER)]

        barrier_sem = pltpu.get_barrier_semaphore()
        for q in peers:
            pl.semaphore_signal(
                barrier_sem, inc=1,
                device_id=(q,), device_id_type=pl.DeviceIdType.MESH,
            )
        pl.semaphore_wait(barrier_sem, N_PEER)

        xf = x_ref[:, :]
        scores = jnp.dot(xf, rw_ref[:, :], preferred_element_type=jnp.float32)
        s_max = jnp.max(scores, axis=-1, keepdims=True)
        p = jnp.exp(scores - s_max)
        probs = p / jnp.sum(p, axis=-1, keepdims=True)

        idx0 = idx_ref[:, 0:1]
        idx1 = idx_ref[:, 1:2]
        eids = lax.broadcasted_iota(jnp.int32, (N_TOK, N_EXP), 1)
        g0 = jnp.sum(jnp.where(eids == idx0, probs, 0.0), axis=1, keepdims=True)
        g1 = jnp.sum(jnp.where(eids == idx1, probs, 0.0), axis=1, keepdims=True)
        gs = g0 + g1
        g0 = g0 / gs
        g1 = g1 / gs
        w_cols = []
        for j in range(E_LOCAL):
            e = my * E_LOCAL + j
            w_cols.append(jnp.where(idx0 == e, g0, 0.0)
                          + jnp.where(idx1 == e, g1, 0.0))
        w_ref[:, :] = jnp.concatenate(w_cols, axis=1)

        ewb_ref[...] = ew_ref[...].astype(jnp.bfloat16)
        ewcat = ewb_ref[...].reshape(E_LOCAL * D_IN, D_OUT)

        def partial_chunk(c):
            rows = pl.ds(c * CHUNK, CHUNK)
            xc = x_ref[rows, :]
            wc = w_ref[rows, :]
            xcat = jnp.concatenate(
                [(wc[:, j:j + 1] * xc).astype(jnp.bfloat16)
                 for j in range(E_LOCAL)],
                axis=1,
            )
            return jnp.dot(xcat, ewcat, preferred_element_type=jnp.float32)

        def rdma(src, dst, ssem, rsem, dev):
            return pltpu.make_async_remote_copy(
                src_ref=src, dst_ref=dst, send_sem=ssem, recv_sem=rsem,
                device_id=(dev,), device_id_type=pl.DeviceIdType.MESH,
            )

        rs = []
        for k, q in enumerate(peers):
            rs_send_buf[k] = partial_chunk(q).astype(jnp.bfloat16)
            for h in range(2):
                r = rdma(rs_send_buf.at[k, pl.ds(h * HALF, HALF), :],
                         rs_recv_buf.at[2 - k, pl.ds(h * HALF, HALF), :],
                         rs_send_sems.at[2 * k + h],
                         rs_recv_sems.at[2 * (2 - k) + h],
                         q)
                r.start()
                rs.append(r)

        mine = partial_chunk(my)

        ag = []
        for h in range(2):
            my_rows = pl.ds(my * CHUNK + h * HALF, HALF)
            for j in range(N_PEER):
                rdma(rs_send_buf.at[j, pl.ds(h * HALF, HALF), :],
                     rs_recv_buf.at[j, pl.ds(h * HALF, HALF), :],
                     rs_send_sems.at[2 * j + h],
                     rs_recv_sems.at[2 * j + h],
                     my).wait_recv()
            red = mine[h * HALF:(h + 1) * HALF, :]
            for j in range(N_PEER):
                red = red + rs_recv_buf[j, pl.ds(h * HALF, HALF), :].astype(
                    jnp.float32)
            out_ref[my_rows, :] = red.astype(jnp.bfloat16)
            for k, q in enumerate(peers):
                r = rdma(out_ref.at[my_rows, :],
                         out_ref.at[my_rows, :],
                         ag_send_sems.at[2 * k + h],
                         ag_recv_sems.at[2 * (2 - k) + h],
                         q)
                r.start()
                ag.append(r)
        for r in rs:
            r.wait_send()

        for j in range(N_PEER):
            u = _mod(my + j + 1)
            for h in range(2):
                rows = pl.ds(u * CHUNK + h * HALF, HALF)
                rdma(out_ref.at[rows, :],
                     out_ref.at[rows, :],
                     ag_send_sems.at[2 * j + h],
                     ag_recv_sems.at[2 * j + h],
                     my).wait_recv()
        for r in ag:
            r.wait_send()

    return pl.pallas_call(
        body,
        out_shape=jax.ShapeDtypeStruct((N_TOK, D_OUT), jnp.bfloat16),
        in_specs=[
            pl.BlockSpec(memory_space=pltpu.VMEM),
            pl.BlockSpec(memory_space=pltpu.VMEM),
            pl.BlockSpec(memory_space=pltpu.VMEM),
            pl.BlockSpec(memory_space=pltpu.VMEM),
        ],
        out_specs=pl.BlockSpec(memory_space=pltpu.VMEM),
        scratch_shapes=[
            pltpu.VMEM((E_LOCAL, D_IN, D_OUT), jnp.bfloat16),
            pltpu.VMEM((N_TOK, E_LOCAL), jnp.float32),
            pltpu.VMEM((N_PEER, CHUNK, D_OUT), jnp.bfloat16),
            pltpu.VMEM((N_PEER, CHUNK, D_OUT), jnp.bfloat16),
            pltpu.SemaphoreType.DMA((2 * N_PEER,)),
            pltpu.SemaphoreType.DMA((2 * N_PEER,)),
            pltpu.SemaphoreType.DMA((2 * N_PEER,)),
            pltpu.SemaphoreType.DMA((2 * N_PEER,)),
        ],
        compiler_params=pltpu.CompilerParams(collective_id=0),
    )(x, router_W, route_idx, expert_W)


# device time: 17999 ns/iter; 2.0605x vs baseline; 1.0124x over previous
import jax
import jax.numpy as jnp
from jax import lax
from jax.experimental import pallas as pl
from jax.experimental.pallas import tpu as pltpu

N_DEV = 4
N_TOK = 512
D_IN = 256
D_OUT = 512
N_EXP = 16
E_LOCAL = N_EXP // N_DEV
CHUNK = N_TOK // N_DEV
HALF = CHUNK // 2
N_PEER = N_DEV - 1


def _mod(v):
    return lax.rem(v + 2 * N_DEV, N_DEV)


def kernel(x, router_W, route_idx, expert_W):
    def body(x_ref, rw_ref, idx_ref, ew_ref, out_ref,
             ewb_ref, w_ref, rs_send_buf, rs_recv_buf,
             rs_send_sems, rs_recv_sems, ag_send_sems, ag_recv_sems):
        my = lax.axis_index("i")
        peers = [_mod(my + k + 1) for k in range(N_PEER)]

        barrier_sem = pltpu.get_barrier_semaphore()
        for q in peers:
            pl.semaphore_signal(
                barrier_sem, inc=1,
                device_id=(q,), device_id_type=pl.DeviceIdType.MESH,
            )

        xf = x_ref[:, :]
        scores = jnp.dot(xf, rw_ref[:, :], preferred_element_type=jnp.float32)
        s_max = jnp.max(scores, axis=-1, keepdims=True)
        p = jnp.exp(scores - s_max)
        probs = p / jnp.sum(p, axis=-1, keepdims=True)

        idx0 = idx_ref[:, 0:1]
        idx1 = idx_ref[:, 1:2]
        eids = lax.broadcasted_iota(jnp.int32, (N_TOK, N_EXP), 1)
        g0 = jnp.sum(jnp.where(eids == idx0, probs, 0.0), axis=1, keepdims=True)
        g1 = jnp.sum(jnp.where(eids == idx1, probs, 0.0), axis=1, keepdims=True)
        gs = g0 + g1
        g0 = g0 / gs
        g1 = g1 / gs
        w_cols = []
        for j in range(E_LOCAL):
            e = my * E_LOCAL + j
            w_cols.append(jnp.where(idx0 == e, g0, 0.0)
                          + jnp.where(idx1 == e, g1, 0.0))
        w_ref[:, :] = jnp.concatenate(w_cols, axis=1)

        ewb_ref[...] = ew_ref[...].astype(jnp.bfloat16)
        ewcat = ewb_ref[...].reshape(E_LOCAL * D_IN, D_OUT)

        def partial_chunk(c):
            rows = pl.ds(c * CHUNK, CHUNK)
            xc = x_ref[rows, :]
            wc = w_ref[rows, :]
            xcat = jnp.concatenate(
                [(wc[:, j:j + 1] * xc).astype(jnp.bfloat16)
                 for j in range(E_LOCAL)],
                axis=1,
            )
            return jnp.dot(xcat, ewcat, preferred_element_type=jnp.float32)

        def rdma(src, dst, ssem, rsem, dev):
            return pltpu.make_async_remote_copy(
                src_ref=src, dst_ref=dst, send_sem=ssem, recv_sem=rsem,
                device_id=(dev,), device_id_type=pl.DeviceIdType.MESH,
            )

        rs = []
        for k, q in enumerate(peers):
            rs_send_buf[k] = partial_chunk(q).astype(jnp.bfloat16)
            if k == 0:
                pl.semaphore_wait(barrier_sem, N_PEER)
            for h in range(2):
                r = rdma(rs_send_buf.at[k, pl.ds(h * HALF, HALF), :],
                         rs_recv_buf.at[2 - k, pl.ds(h * HALF, HALF), :],
                         rs_send_sems.at[2 * k + h],
                         rs_recv_sems.at[2 * (2 - k) + h],
                         q)
                r.start()
                rs.append(r)

        mine = partial_chunk(my)

        ag = []
        for h in range(2):
            my_rows = pl.ds(my * CHUNK + h * HALF, HALF)
            for j in range(N_PEER):
                rdma(rs_send_buf.at[j, pl.ds(h * HALF, HALF), :],
                     rs_recv_buf.at[j, pl.ds(h * HALF, HALF), :],
                     rs_send_sems.at[2 * j + h],
                     rs_recv_sems.at[2 * j + h],
                     my).wait_recv()
            red = mine[h * HALF:(h + 1) * HALF, :]
            for j in range(N_PEER):
                red = red + rs_recv_buf[j, pl.ds(h * HALF, HALF), :].astype(
                    jnp.float32)
            out_ref[my_rows, :] = red.astype(jnp.bfloat16)
            for k, q in enumerate(peers):
                r = rdma(out_ref.at[my_rows, :],
                         out_ref.at[my_rows, :],
                         ag_send_sems.at[2 * k + h],
                         ag_recv_sems.at[2 * (2 - k) + h],
                         q)
                r.start()
                ag.append(r)
        for r in rs:
            r.wait_send()

        for j in range(N_PEER):
            u = _mod(my + j + 1)
            for h in range(2):
                rows = pl.ds(u * CHUNK + h * HALF, HALF)
                rdma(out_ref.at[rows, :],
                     out_ref.at[rows, :],
                     ag_send_sems.at[2 * j + h],
                     ag_recv_sems.at[2 * j + h],
                     my).wait_recv()
        for r in ag:
            r.wait_send()

    return pl.pallas_call(
        body,
        out_shape=jax.ShapeDtypeStruct((N_TOK, D_OUT), jnp.bfloat16),
        in_specs=[
            pl.BlockSpec(memory_space=pltpu.VMEM),
            pl.BlockSpec(memory_space=pltpu.VMEM),
            pl.BlockSpec(memory_space=pltpu.VMEM),
            pl.BlockSpec(memory_space=pltpu.VMEM),
        ],
        out_specs=pl.BlockSpec(memory_space=pltpu.VMEM),
        scratch_shapes=[
            pltpu.VMEM((E_LOCAL, D_IN, D_OUT), jnp.bfloat16),
            pltpu.VMEM((N_TOK, E_LOCAL), jnp.float32),
            pltpu.VMEM((N_PEER, CHUNK, D_OUT), jnp.bfloat16),
            pltpu.VMEM((N_PEER, CHUNK, D_OUT), jnp.bfloat16),
            pltpu.SemaphoreType.DMA((2 * N_PEER,)),
            pltpu.SemaphoreType.DMA((2 * N_PEER,)),
            pltpu.SemaphoreType.DMA((2 * N_PEER,)),
            pltpu.SemaphoreType.DMA((2 * N_PEER,)),
        ],
        compiler_params=pltpu.CompilerParams(collective_id=0),
    )(x, router_W, route_idx, expert_W)
